# baseline (device time: 348727 ns/iter reference)
import numpy as np

import jax
import jax.numpy as jnp
from jax import lax
from jax.experimental import pallas as pl
from jax.experimental.pallas import tpu as pltpu

N_DEV = 4
SQ = 2048
DM = 1024
HQ = 8
DH = 128
HD = HQ * DH
CHUNK = SQ // N_DEV
SCALE = 0.08838834764831843

_BLOCKS = [b for r in range(3) for b in range(SQ // 64) if b % 3 == r]
_PERM = np.concatenate([np.arange(64 * b, 64 * b + 64) for b in _BLOCKS])
_INV_PERM = np.argsort(_PERM)
_N0, _N1, _N2 = 704, 704, 640


def kernel(x, Wq, K_ext, V_ext, Wo):
    my = lax.axis_index("i")
    perm = jnp.asarray(_PERM)
    xb = x[0][perm].astype(jnp.bfloat16)
    wq = (lax.dynamic_slice_in_dim(Wq, my * HD, HD, 1) * SCALE).astype(jnp.bfloat16)
    wo = lax.dynamic_slice_in_dim(Wo, my * HD, HD, 0).astype(jnp.bfloat16)
    wq3 = wq.reshape(DM, HQ, DH).transpose(1, 0, 2)
    wo3 = wo.reshape(HQ, DH, DM)
    k3 = K_ext[0].transpose(1, 0, 2)[:, perm, :].astype(jnp.bfloat16)
    v3 = V_ext[0].transpose(1, 0, 2)[:, perm, :].astype(jnp.bfloat16)

    def body(x_ref, wq_ref, k_ref, v_ref, wo_ref, out_ref,
             acc_ref, sbuf_ref, rbuf_ref, send_sems, recv_sems):
        p = lax.axis_index("i")
        left = (p - 1 + N_DEV) % N_DEV
        right = (p + 1) % N_DEV

        barrier_sem = pltpu.get_barrier_semaphore()
        for nbr in (left, right):
            pl.semaphore_signal(barrier_sem, inc=1, device_id=(nbr,),
                                device_id_type=pl.DeviceIdType.MESH)
        pl.semaphore_wait(barrier_sem, 2)

        def tdot(a, b):
            return lax.dot_general(a, b, (((1,), (1,)), ((), ())),
                                   preferred_element_type=jnp.float32)

        def bdot_t(a, b):
            return lax.dot_general(a, b, (((2,), (2,)), ((0,), (0,))),
                                   preferred_element_type=jnp.float32)

        def bdot(a, b):
            return lax.dot_general(a, b, (((2,), (1,)), ((0,), (0,))),
                                   preferred_element_type=jnp.float32)

        def attn_class(q0, n_rows, main_lo, main_n, diag_lo, n_blk):
            def h_loop(h, acc_t):
                q = jnp.dot(x_ref[pl.ds(q0, n_rows), :], wq_ref[h, :, :],
                            preferred_element_type=jnp.float32
                            ).astype(jnp.bfloat16)
                if main_n == 0:
                    w = jnp.exp(tdot(q, k_ref[h, pl.ds(q0, n_rows), :]))
                    denom = jnp.sum(w, axis=-1, keepdims=True)
                    ctx = jnp.dot(w.astype(jnp.bfloat16),
                                  v_ref[h, pl.ds(q0, n_rows), :],
                                  preferred_element_type=jnp.float32)
                else:
                    wm = jnp.exp(tdot(q, k_ref[h, pl.ds(main_lo, main_n), :]))
                    wb = jnp.exp(tdot(q, k_ref[h, 0:64, :]))
                    qd = q.reshape(n_blk, 64, DH)
                    wd = jnp.exp(bdot_t(qd, k_ref[h, pl.ds(diag_lo, n_rows), :]
                                        .reshape(n_blk, 64, DH)))
                    denom = (jnp.sum(wm, axis=-1, keepdims=True)
                             + jnp.sum(wb, axis=-1, keepdims=True)
                             + jnp.sum(wd, axis=-1).reshape(n_rows, 1))
                    ctx = (jnp.dot(wm.astype(jnp.bfloat16),
                                   v_ref[h, pl.ds(main_lo, main_n), :],
                                   preferred_element_type=jnp.float32)
                           + jnp.dot(wb.astype(jnp.bfloat16), v_ref[h, 0:64, :],
                                     preferred_element_type=jnp.float32)
                           + bdot(wd.astype(jnp.bfloat16),
                                  v_ref[h, pl.ds(diag_lo, n_rows), :]
                                  .reshape(n_blk, 64, DH)).reshape(n_rows, DH))
                ctx = (ctx / denom).astype(jnp.bfloat16)
                return acc_t + jnp.dot(ctx, wo_ref[h, :, :],
                                       preferred_element_type=jnp.float32)

            acc_c = lax.fori_loop(0, HQ, h_loop,
                                  jnp.zeros((n_rows, DM), jnp.float32))
            acc_ref[pl.ds(q0, n_rows), :] = acc_c

        attn_class(0, _N0, 0, 0, 0, _N0 // 64)
        attn_class(_N0, _N1, _N0 + _N1, _N2, _N0, _N1 // 64)
        attn_class(_N0 + _N1, _N2, _N0, _N1, _N0 + _N1, _N2 // 64)

        def rows(i):
            return pl.ds(i * CHUNK, CHUNK)

        def ring_rdma(src, slot):
            return pltpu.make_async_remote_copy(
                src_ref=src, dst_ref=rbuf_ref.at[slot],
                send_sem=send_sems.at[slot], recv_sem=recv_sems.at[slot],
                device_id=(right,), device_id_type=pl.DeviceIdType.MESH)

        for h in range(N_DEV - 1):
            s_idx = (p - h + N_DEV) % N_DEV
            payload = acc_ref[rows(s_idx), :]
            if h > 0:
                payload = payload + rbuf_ref[h - 1].astype(jnp.float32)
            sbuf_ref[h, :, :] = payload.astype(jnp.bfloat16)
            rdma = ring_rdma(sbuf_ref.at[h], h)
            rdma.start()
            rdma.wait()

        g = (p + 1) % N_DEV
        full = acc_ref[rows(g), :] + rbuf_ref[N_DEV - 2].astype(jnp.float32)
        out_ref[rows(g), :] = full
        sbuf_ref[N_DEV - 1, :, :] = full.astype(jnp.bfloat16)

        for h in range(N_DEV - 1, 2 * (N_DEV - 1)):
            src = sbuf_ref.at[N_DEV - 1] if h == N_DEV - 1 else rbuf_ref.at[h - 1]
            rdma = ring_rdma(src, h)
            rdma.start()
            rdma.wait()
            o_idx = (p - (h - (N_DEV - 1)) + N_DEV) % N_DEV
            out_ref[rows(o_idx), :] = rbuf_ref[h].astype(jnp.float32)

    out = pl.pallas_call(
        body,
        out_shape=jax.ShapeDtypeStruct((SQ, DM), jnp.float32),
        in_specs=[pl.BlockSpec(memory_space=pltpu.VMEM)] * 5,
        out_specs=pl.BlockSpec(memory_space=pltpu.VMEM),
        scratch_shapes=[
            pltpu.VMEM((SQ, DM), jnp.float32),
            pltpu.VMEM((N_DEV, CHUNK, DM), jnp.bfloat16),
            pltpu.VMEM((2 * (N_DEV - 1), CHUNK, DM), jnp.bfloat16),
            pltpu.SemaphoreType.DMA((2 * (N_DEV - 1),)),
            pltpu.SemaphoreType.DMA((2 * (N_DEV - 1),)),
        ],
        compiler_params=pltpu.CompilerParams(
            collective_id=0, vmem_limit_bytes=100 * 1024 * 1024),
    )(xb, wq3, k3, v3, wo3)
    return out[jnp.asarray(_INV_PERM)][None]


# device time: 201478 ns/iter; 1.7308x vs baseline; 1.7308x over previous
import numpy as np

import jax
import jax.numpy as jnp
from jax import lax
from jax.experimental import pallas as pl
from jax.experimental.pallas import tpu as pltpu

N_DEV = 4
SQ = 2048
DM = 1024
HQ = 8
DH = 128
HD = HQ * DH
CHUNK = SQ // N_DEV
SCALE = 0.08838834764831843

_N0, _N1, _N2 = 704, 704, 640
_NB = SQ // 64


def _permute_blocks(a):
    ar = a.reshape((_NB, 64) + a.shape[1:])
    return jnp.concatenate([ar[0::3], ar[1::3], ar[2::3]], axis=0).reshape(a.shape)


def _unpermute_blocks(a):
    ar = a.reshape((_NB, 64) + a.shape[1:])
    g0, g1, g2 = ar[0:11], ar[11:22], ar[22:32]
    head = jnp.stack([g0[:10], g1[:10], g2[:10]], axis=1)
    head = head.reshape((30 * 64,) + a.shape[1:])
    tail = jnp.concatenate([g0[10:11], g1[10:11]], axis=0)
    tail = tail.reshape((2 * 64,) + a.shape[1:])
    return jnp.concatenate([head, tail], axis=0)


def kernel(x, Wq, K_ext, V_ext, Wo):
    my = lax.axis_index("i")
    xb = _permute_blocks(x[0].astype(jnp.bfloat16))
    wq = (lax.dynamic_slice_in_dim(Wq, my * HD, HD, 1) * SCALE).astype(jnp.bfloat16)
    wo = lax.dynamic_slice_in_dim(Wo, my * HD, HD, 0).astype(jnp.bfloat16)
    wq3 = wq.reshape(DM, HQ, DH).transpose(1, 0, 2)
    wo3 = wo.reshape(HQ, DH, DM)
    k3 = _permute_blocks(K_ext[0].astype(jnp.bfloat16)).transpose(1, 0, 2)
    v3 = _permute_blocks(V_ext[0].astype(jnp.bfloat16)).transpose(1, 0, 2)

    def body(x_ref, wq_ref, k_ref, v_ref, wo_ref, out_ref,
             acc_ref, sbuf_ref, rbuf_ref, send_sems, recv_sems):
        p = lax.axis_index("i")
        left = (p - 1 + N_DEV) % N_DEV
        right = (p + 1) % N_DEV

        barrier_sem = pltpu.get_barrier_semaphore()
        for nbr in (left, right):
            pl.semaphore_signal(barrier_sem, inc=1, device_id=(nbr,),
                                device_id_type=pl.DeviceIdType.MESH)
        pl.semaphore_wait(barrier_sem, 2)

        def tdot(a, b):
            return lax.dot_general(a, b, (((1,), (1,)), ((), ())),
                                   preferred_element_type=jnp.float32)

        def bdot_t(a, b):
            return lax.dot_general(a, b, (((2,), (2,)), ((0,), (0,))),
                                   preferred_element_type=jnp.float32)

        def bdot(a, b):
            return lax.dot_general(a, b, (((2,), (1,)), ((0,), (0,))),
                                   preferred_element_type=jnp.float32)

        def attn_class(q0, n_rows, main_lo, main_n, diag_lo, n_blk):
            def h_loop(h, acc_t):
                q = jnp.dot(x_ref[pl.ds(q0, n_rows), :], wq_ref[h, :, :],
                            preferred_element_type=jnp.float32
                            ).astype(jnp.bfloat16)
                if main_n == 0:
                    w = jnp.exp(tdot(q, k_ref[h, pl.ds(q0, n_rows), :]))
                    denom = jnp.sum(w, axis=-1, keepdims=True)
                    ctx = jnp.dot(w.astype(jnp.bfloat16),
                                  v_ref[h, pl.ds(q0, n_rows), :],
                                  preferred_element_type=jnp.float32)
                else:
                    wm = jnp.exp(tdot(q, k_ref[h, pl.ds(main_lo, main_n), :]))
                    wb = jnp.exp(tdot(q, k_ref[h, 0:64, :]))
                    qd = q.reshape(n_blk, 64, DH)
                    wd = jnp.exp(bdot_t(qd, k_ref[h, pl.ds(diag_lo, n_rows), :]
                                        .reshape(n_blk, 64, DH)))
                    denom = (jnp.sum(wm, axis=-1, keepdims=True)
                             + jnp.sum(wb, axis=-1, keepdims=True)
                             + jnp.sum(wd, axis=-1).reshape(n_rows, 1))
                    ctx = (jnp.dot(wm.astype(jnp.bfloat16),
                                   v_ref[h, pl.ds(main_lo, main_n), :],
                                   preferred_element_type=jnp.float32)
                           + jnp.dot(wb.astype(jnp.bfloat16), v_ref[h, 0:64, :],
                                     preferred_element_type=jnp.float32)
                           + bdot(wd.astype(jnp.bfloat16),
                                  v_ref[h, pl.ds(diag_lo, n_rows), :]
                                  .reshape(n_blk, 64, DH)).reshape(n_rows, DH))
                ctx = (ctx / denom).astype(jnp.bfloat16)
                return acc_t + jnp.dot(ctx, wo_ref[h, :, :],
                                       preferred_element_type=jnp.float32)

            acc_c = lax.fori_loop(0, HQ, h_loop,
                                  jnp.zeros((n_rows, DM), jnp.float32))
            acc_ref[pl.ds(q0, n_rows), :] = acc_c

        attn_class(0, _N0, 0, 0, 0, _N0 // 64)
        attn_class(_N0, _N1, _N0 + _N1, _N2, _N0, _N1 // 64)
        attn_class(_N0 + _N1, _N2, _N0, _N1, _N0 + _N1, _N2 // 64)

        def rows(i):
            return pl.ds(i * CHUNK, CHUNK)

        def ring_rdma(src, slot):
            return pltpu.make_async_remote_copy(
                src_ref=src, dst_ref=rbuf_ref.at[slot],
                send_sem=send_sems.at[slot], recv_sem=recv_sems.at[slot],
                device_id=(right,), device_id_type=pl.DeviceIdType.MESH)

        for h in range(N_DEV - 1):
            s_idx = (p - h + N_DEV) % N_DEV
            payload = acc_ref[rows(s_idx), :]
            if h > 0:
                payload = payload + rbuf_ref[h - 1].astype(jnp.float32)
            sbuf_ref[h, :, :] = payload.astype(jnp.bfloat16)
            rdma = ring_rdma(sbuf_ref.at[h], h)
            rdma.start()
            rdma.wait()

        g = (p + 1) % N_DEV
        full = acc_ref[rows(g), :] + rbuf_ref[N_DEV - 2].astype(jnp.float32)
        out_ref[rows(g), :] = full
        sbuf_ref[N_DEV - 1, :, :] = full.astype(jnp.bfloat16)

        for h in range(N_DEV - 1, 2 * (N_DEV - 1)):
            src = sbuf_ref.at[N_DEV - 1] if h == N_DEV - 1 else rbuf_ref.at[h - 1]
            rdma = ring_rdma(src, h)
            rdma.start()
            rdma.wait()
            o_idx = (p - (h - (N_DEV - 1)) + N_DEV) % N_DEV
            out_ref[rows(o_idx), :] = rbuf_ref[h].astype(jnp.float32)

    out = pl.pallas_call(
        body,
        out_shape=jax.ShapeDtypeStruct((SQ, DM), jnp.float32),
        in_specs=[pl.BlockSpec(memory_space=pltpu.VMEM)] * 5,
        out_specs=pl.BlockSpec(memory_space=pltpu.VMEM),
        scratch_shapes=[
            pltpu.VMEM((SQ, DM), jnp.float32),
            pltpu.VMEM((N_DEV, CHUNK, DM), jnp.bfloat16),
            pltpu.VMEM((2 * (N_DEV - 1), CHUNK, DM), jnp.bfloat16),
            pltpu.SemaphoreType.DMA((2 * (N_DEV - 1),)),
            pltpu.SemaphoreType.DMA((2 * (N_DEV - 1),)),
        ],
        compiler_params=pltpu.CompilerParams(
            collective_id=0, vmem_limit_bytes=100 * 1024 * 1024),
    )(xb, wq3, k3, v3, wo3)
    return _unpermute_blocks(out)[None]


# device time: 173592 ns/iter; 2.0089x vs baseline; 1.1606x over previous
import jax
import jax.numpy as jnp
from jax import lax
from jax.experimental import pallas as pl
from jax.experimental.pallas import tpu as pltpu

N_DEV = 4
SQ = 2048
DM = 1024
HQ = 8
DH = 128
HD = HQ * DH
CHUNK = SQ // N_DEV
SCALE = 0.08838834764831843

_NB = SQ // 64
_BLOCKS = [b for r in range(3) for b in range(_NB) if b % 3 == r]
_N0, _N1, _N2 = 704, 704, 640


def kernel(x, Wq, K_ext, V_ext, Wo):
    my = lax.axis_index("i")
    xb = x[0].astype(jnp.bfloat16)
    wq = (lax.dynamic_slice_in_dim(Wq, my * HD, HD, 1) * SCALE).astype(jnp.bfloat16)
    wo = lax.dynamic_slice_in_dim(Wo, my * HD, HD, 0).astype(jnp.bfloat16)
    wq3 = wq.reshape(DM, HQ, DH).transpose(1, 0, 2)
    wo3 = wo.reshape(HQ, DH, DM)
    k3 = K_ext[0].transpose(1, 0, 2).astype(jnp.bfloat16)
    v3 = V_ext[0].transpose(1, 0, 2).astype(jnp.bfloat16)

    def body(x_ref, wq_ref, k_ref, v_ref, wo_ref, out_ref,
             xp_ref, kp_ref, vp_ref, sbuf_ref, rbuf_ref, send_sems, recv_sems):
        p = lax.axis_index("i")
        left = (p - 1 + N_DEV) % N_DEV
        right = (p + 1) % N_DEV

        barrier_sem = pltpu.get_barrier_semaphore()
        for nbr in (left, right):
            pl.semaphore_signal(barrier_sem, inc=1, device_id=(nbr,),
                                device_id_type=pl.DeviceIdType.MESH)
        pl.semaphore_wait(barrier_sem, 2)

        for j, b in enumerate(_BLOCKS):
            xp_ref[j * 64:(j + 1) * 64, :] = x_ref[b * 64:(b + 1) * 64, :]
            kp_ref[:, j * 64:(j + 1) * 64, :] = k_ref[:, b * 64:(b + 1) * 64, :]
            vp_ref[:, j * 64:(j + 1) * 64, :] = v_ref[:, b * 64:(b + 1) * 64, :]

        def tdot(a, b):
            return lax.dot_general(a, b, (((1,), (1,)), ((), ())),
                                   preferred_element_type=jnp.float32)

        def bdot_t(a, b):
            return lax.dot_general(a, b, (((2,), (2,)), ((0,), (0,))),
                                   preferred_element_type=jnp.float32)

        def bdot(a, b):
            return lax.dot_general(a, b, (((2,), (1,)), ((0,), (0,))),
                                   preferred_element_type=jnp.float32)

        def attn_part(q_lo, n_rows, main_lo, main_n):
            n_blk = n_rows // 64

            def h_loop(h, acc_t):
                q = jnp.dot(xp_ref[pl.ds(q_lo, n_rows), :], wq_ref[h, :, :],
                            preferred_element_type=jnp.float32
                            ).astype(jnp.bfloat16)
                if main_n == 0:
                    w = jnp.exp(tdot(q, kp_ref[h, 0:_N0, :]))
                    denom = jnp.sum(w, axis=-1, keepdims=True)
                    ctx = jnp.dot(w.astype(jnp.bfloat16), vp_ref[h, 0:_N0, :],
                                  preferred_element_type=jnp.float32)
                else:
                    wm = jnp.exp(tdot(q, kp_ref[h, pl.ds(main_lo, main_n), :]))
                    wb = jnp.exp(tdot(q, kp_ref[h, 0:64, :]))
                    qd = q.reshape(n_blk, 64, DH)
                    wd = jnp.exp(bdot_t(qd, kp_ref[h, pl.ds(q_lo, n_rows), :]
                                        .reshape(n_blk, 64, DH)))
                    denom = (jnp.sum(wm, axis=-1, keepdims=True)
                             + jnp.sum(wb, axis=-1, keepdims=True)
                             + jnp.sum(wd, axis=-1).reshape(n_rows, 1))
                    ctx = (jnp.dot(wm.astype(jnp.bfloat16),
                                   vp_ref[h, pl.ds(main_lo, main_n), :],
                                   preferred_element_type=jnp.float32)
                           + jnp.dot(wb.astype(jnp.bfloat16), vp_ref[h, 0:64, :],
                                     preferred_element_type=jnp.float32)
                           + bdot(wd.astype(jnp.bfloat16),
                                  vp_ref[h, pl.ds(q_lo, n_rows), :]
                                  .reshape(n_blk, 64, DH)).reshape(n_rows, DH))
                ctx = (ctx / denom).astype(jnp.bfloat16)
                return acc_t + jnp.dot(ctx, wo_ref[h, :, :],
                                       preferred_element_type=jnp.float32)

            return lax.fori_loop(0, HQ, h_loop,
                                 jnp.zeros((n_rows, DM), jnp.float32))

        def chunk0():
            return attn_part(0, 512, 0, 0)

        def chunk1():
            return jnp.concatenate(
                [attn_part(512, 192, 0, 0),
                 attn_part(704, 320, _N0 + _N1, _N2)], axis=0)

        def chunk2():
            return jnp.concatenate(
                [attn_part(1024, 384, _N0 + _N1, _N2),
                 attn_part(1408, 128, _N0, _N1)], axis=0)

        def chunk3():
            return attn_part(1536, 512, _N0, _N1)

        def nat_block(pb):
            return lax.select(pb < 11, 3 * pb,
                              lax.select(pb < 22, 3 * (pb - 11) + 1,
                                         3 * (pb - 22) + 2))

        def scatter_chunk_value(c_idx, val):
            for j in range(CHUNK // 64):
                nb = nat_block(8 * c_idx + j)
                out_ref[pl.ds(nb * 64, 64), :] = val[j * 64:(j + 1) * 64, :]

        def ring_rdma(src, slot):
            return pltpu.make_async_remote_copy(
                src_ref=src, dst_ref=rbuf_ref.at[slot],
                send_sem=send_sems.at[slot], recv_sem=recv_sems.at[slot],
                device_id=(right,), device_id_type=pl.DeviceIdType.MESH)

        rs = [None] * (N_DEV - 1)
        for h in range(N_DEV):
            c_idx = (p - h + N_DEV) % N_DEV
            payload = lax.switch(c_idx, [chunk0, chunk1, chunk2, chunk3])
            if h > 0:
                rs[h - 1].wait()
                payload = payload + rbuf_ref[h - 1].astype(jnp.float32)
            if h < N_DEV - 1:
                sbuf_ref[h, :, :] = payload.astype(jnp.bfloat16)
                rs[h] = ring_rdma(sbuf_ref.at[h], h)
                rs[h].start()
            else:
                scatter_chunk_value(c_idx, payload)
                sbuf_ref[N_DEV - 1, :, :] = payload.astype(jnp.bfloat16)

        for h in range(N_DEV - 1, 2 * (N_DEV - 1)):
            src = sbuf_ref.at[N_DEV - 1] if h == N_DEV - 1 else rbuf_ref.at[h - 1]
            rdma = ring_rdma(src, h)
            rdma.start()
            rdma.wait()
            o_idx = (p - (h - (N_DEV - 1)) + N_DEV) % N_DEV
            for j in range(CHUNK // 64):
                nb = nat_block(8 * o_idx + j)
                out_ref[pl.ds(nb * 64, 64), :] = (
                    rbuf_ref[h, j * 64:(j + 1) * 64, :].astype(jnp.float32))

    out = pl.pallas_call(
        body,
        out_shape=jax.ShapeDtypeStruct((SQ, DM), jnp.float32),
        in_specs=[pl.BlockSpec(memory_space=pltpu.VMEM)] * 5,
        out_specs=pl.BlockSpec(memory_space=pltpu.VMEM),
        scratch_shapes=[
            pltpu.VMEM((SQ, DM), jnp.bfloat16),
            pltpu.VMEM((HQ, SQ, DH), jnp.bfloat16),
            pltpu.VMEM((HQ, SQ, DH), jnp.bfloat16),
            pltpu.VMEM((N_DEV, CHUNK, DM), jnp.bfloat16),
            pltpu.VMEM((2 * (N_DEV - 1), CHUNK, DM), jnp.bfloat16),
            pltpu.SemaphoreType.DMA((2 * (N_DEV - 1),)),
            pltpu.SemaphoreType.DMA((2 * (N_DEV - 1),)),
        ],
        compiler_params=pltpu.CompilerParams(
            collective_id=0, vmem_limit_bytes=100 * 1024 * 1024),
    )(xb, wq3, k3, v3, wo3)
    return out[None]
